# baseline (speedup 1.0000x reference)
import functools

import jax
import jax.numpy as jnp
from jax import lax
from jax.experimental import pallas as pl
from jax.experimental.pallas import tpu as pltpu
from jax.experimental.pallas import tpu_sc as plsc

NC = 2
NS = 16
NW = NC * NS
L = 16
D = 64
BT = 128


def _sc_gather_tiles(idx_re, tab_f, batch, hist):
  bt_per_w = batch // BT // NW
  ET = D // 8

  mesh = plsc.VectorSubcoreMesh(
      core_axis_name="c", subcore_axis_name="s", num_cores=NC, num_subcores=NS
  )

  @functools.partial(
      pl.kernel,
      out_type=jax.ShapeDtypeStruct((hist, ET, batch // BT, 8, BT),
                                    jnp.float32),
      mesh=mesh,
      scratch_types=[
          pltpu.VMEM((BT * hist,), jnp.int32),
          pltpu.VMEM((BT, D), jnp.float32),
          pltpu.VMEM((BT, D), jnp.float32),
          pltpu.VMEM((ET, 8, BT), jnp.float32),
          pltpu.VMEM((ET, 8, BT), jnp.float32),
          pltpu.SemaphoreType.DMA,
          pltpu.SemaphoreType.DMA,
          pltpu.SemaphoreType.DMA,
          pltpu.SemaphoreType.DMA,
      ],
      compiler_params=pltpu.CompilerParams(
          use_tc_tiling_on_sc=False, needs_layout_passes=False),
  )
  def body(idx_hbm, tab_hbm, out_hbm, idxb_v, rows0, rows1, tile0, tile1,
           semg0, semg1, semo0, semo1):
    wid = lax.axis_index("s") * NC + lax.axis_index("c")
    lanes = jnp.arange(L, dtype=jnp.int32)
    col_hi = [(M * L + lanes) // 8 for M in range(4)]
    col_lo = [(M * L + lanes) % 8 for M in range(4)]
    rows = [rows0, rows1]
    tile = [tile0, tile1]
    semg = [semg0, semg1]
    semo = [semo0, semo1]

    def fire_gather(p, h):
      pltpu.async_copy(tab_hbm.at[idxb_v.at[pl.ds(h * BT, BT)]], rows[p],
                       semg[p])

    def drain_gather(p, h):
      pltpu.make_async_copy(tab_hbm.at[idxb_v.at[pl.ds(h * BT, BT)]], rows[p],
                            semg[p]).wait()

    def fire_out(p, h, bt):
      for et in range(ET):
        pltpu.async_copy(tile[p].at[et], out_hbm.at[h, et, bt], semo[p])

    def drain_out(p, h, bt):
      for et in range(ET):
        pltpu.make_async_copy(tile[p].at[et], out_hbm.at[h, et, bt],
                              semo[p]).wait()

    def blk_body(blk, carry):
      bt = wid * bt_per_w + blk
      off = bt * BT * hist
      pltpu.sync_copy(idx_hbm.at[pl.ds(off, BT * hist)], idxb_v)
      fire_gather(0, 0)

      def h_pair(g, carry2):
        for p in range(2):
          h = 2 * g + p

          @pl.when(h + 1 < hist)
          def _():
            fire_gather(1 - p, h + 1)

          @pl.when(h >= 2)
          def _():
            drain_out(p, h - 2, bt)

          drain_gather(p, h)

          def row_body(r, carry3):
            rsplat = jnp.full((L,), r, dtype=jnp.int32)
            for m in range(4):
              v = rows[p][r, pl.ds(m * L, L)]
              plsc.store_scatter(tile[p], [col_hi[m], col_lo[m], rsplat], v)
            return carry3

          lax.fori_loop(0, BT, row_body, 0)
          fire_out(p, h, bt)
        return carry2

      lax.fori_loop(0, hist // 2, h_pair, 0)
      drain_out(0, hist - 2, bt)
      drain_out(1, hist - 1, bt)
      return carry

    lax.fori_loop(0, bt_per_w, blk_body, 0)

  return body(idx_re, tab_f)


def kernel(indices, q_table, scales):
  batch, hist = indices.shape
  te = q_table[0::2].astype(jnp.float32) * scales[0::2, None]
  to = q_table[1::2].astype(jnp.float32) * scales[1::2, None]
  tab_f = lax.optimization_barrier(jnp.concatenate([te, to], axis=1))
  tab_f = tab_f.reshape(-1, D)
  idx_re = (indices.astype(jnp.int32)
            .reshape(batch // BT, BT, hist)
            .transpose(0, 2, 1)
            .reshape(-1))
  out5 = _sc_gather_tiles(idx_re, tab_f, batch, hist)
  return out5.transpose(2, 4, 0, 1, 3).reshape(batch, hist, D)
